# Initial kernel scaffold; baseline (speedup 1.0000x reference)
import jax
import jax.numpy as jnp
from jax.experimental import pallas as pl


def kernel(inputs, labels, embedding):
    raise NotImplementedError("write your pallas kernel here")


# speedup vs baseline: 1.1283x; 1.1283x over previous
import functools

import jax
import jax.numpy as jnp
from jax import lax
from jax.experimental import pallas as pl
from jax.experimental.pallas import tpu as pltpu
from jax.experimental.pallas import tpu_sc as plsc

D = 32
NC = 2
NS = 16
NW = NC * NS
CHUNK = 800


def _make_kernel(n_rows: int):
    per_w = n_rows // NW
    n_chunks = per_w // CHUNK
    mesh = plsc.VectorSubcoreMesh(core_axis_name="c", subcore_axis_name="s")

    @functools.partial(
        pl.kernel,
        out_type=jax.ShapeDtypeStruct((n_rows, D), jnp.float32),
        mesh=mesh,
        scratch_types=[
            pltpu.VMEM((CHUNK,), jnp.int32),
            pltpu.VMEM((CHUNK, D), jnp.float32),
            pltpu.SemaphoreType.DMA,
        ],
        compiler_params=pltpu.CompilerParams(use_tc_tiling_on_sc=False),
    )
    def run(x_hbm, idx_hbm, emb_hbm, out_hbm, idx_v, buf, sem):
        wid = lax.axis_index("s") * NC + lax.axis_index("c")
        base = wid * per_w

        def body(j, carry):
            off = base + j * CHUNK
            pltpu.sync_copy(idx_hbm.at[pl.ds(off, CHUNK)], idx_v)
            pltpu.sync_copy(x_hbm.at[pl.ds(off, CHUNK)], buf)
            pltpu.async_copy(emb_hbm.at[idx_v], buf, sem, add=True).wait()
            pltpu.sync_copy(buf, out_hbm.at[pl.ds(off, CHUNK)])
            return carry

        lax.fori_loop(0, n_chunks, body, 0)

    return run


def kernel(inputs, labels, embedding):
    b, s, d = inputs.shape
    n = b * s
    x = inputs.reshape(n, d)
    idx = labels.reshape(n).astype(jnp.int32)
    out = _make_kernel(n)(x, idx, embedding)
    return out.reshape(b, s, d)


# speedup vs baseline: 1.1806x; 1.0463x over previous
import functools

import jax
import jax.numpy as jnp
from jax import lax
from jax.experimental import pallas as pl
from jax.experimental.pallas import tpu as pltpu
from jax.experimental.pallas import tpu_sc as plsc

D = 32
NC = 2
NS = 16
NW = NC * NS
CHUNK = 640
NBUF = 4


def _make_kernel(n_rows: int):
    per_w = n_rows // NW
    n_chunks = per_w // CHUNK
    assert per_w % CHUNK == 0 and (n_chunks - NBUF) % NBUF == 0 and n_chunks >= 2 * NBUF
    mesh = plsc.VectorSubcoreMesh(core_axis_name="c", subcore_axis_name="s")

    @functools.partial(
        pl.kernel,
        out_type=jax.ShapeDtypeStruct((n_rows, D), jnp.float32),
        mesh=mesh,
        scratch_types=[
            pltpu.VMEM((per_w,), jnp.int32),
            pltpu.VMEM((NBUF, CHUNK, D), jnp.float32),
            pltpu.SemaphoreType.DMA((NBUF,)),
            pltpu.SemaphoreType.DMA((NBUF,)),
            pltpu.SemaphoreType.DMA((NBUF,)),
        ],
        compiler_params=pltpu.CompilerParams(use_tc_tiling_on_sc=False),
    )
    def run(x_hbm, idx_hbm, emb_hbm, out_hbm, idx_v, bufs, s_ld, s_g, s_st):
        wid = lax.axis_index("s") * NC + lax.axis_index("c")
        base = wid * per_w

        def ld_start(t, b):
            pltpu.async_copy(
                x_hbm.at[pl.ds(base + t * CHUNK, CHUNK)], bufs.at[b], s_ld.at[b])

        def ld_wait(t, b):
            pltpu.make_async_copy(
                x_hbm.at[pl.ds(base + t * CHUNK, CHUNK)], bufs.at[b], s_ld.at[b]).wait()

        def g_start(t, b):
            pltpu.async_copy(
                emb_hbm.at[idx_v.at[pl.ds(t * CHUNK, CHUNK)]], bufs.at[b],
                s_g.at[b], add=True)

        def g_wait(t, b):
            pltpu.make_async_copy(
                emb_hbm.at[idx_v.at[pl.ds(t * CHUNK, CHUNK)]], bufs.at[b],
                s_g.at[b]).wait()

        def st_start(t, b):
            pltpu.async_copy(
                bufs.at[b], out_hbm.at[pl.ds(base + t * CHUNK, CHUNK)], s_st.at[b])

        def st_wait(t, b):
            pltpu.make_async_copy(
                bufs.at[b], out_hbm.at[pl.ds(base + t * CHUNK, CHUNK)], s_st.at[b]).wait()

        pltpu.sync_copy(idx_hbm.at[pl.ds(base, per_w)], idx_v)

        for t in range(NBUF):
            ld_start(t, t)
        ld_wait(0, 0)
        g_start(0, 0)
        ld_wait(1, 1)
        g_start(1, 1)

        T = n_chunks

        def step(t, b):
            g_wait(t, b)
            st_start(t, b)
            st_wait(t, b)

        @pl.loop(0, T - NBUF, step=NBUF)
        def _(j):
            for b in range(NBUF):
                t = j + b
                step(t, b)
                ld_start(t + NBUF, b)
                bn = (b + 2) % NBUF
                ld_wait(t + 2, bn)
                g_start(t + 2, bn)

        for k in range(NBUF):
            t = T - NBUF + k
            b = t % NBUF
            step(t, b)
            tn = t + 2
            if tn <= T - 1:
                bn = tn % NBUF
                ld_wait(tn, bn)
                g_start(tn, bn)

    return run


def kernel(inputs, labels, embedding):
    b, s, d = inputs.shape
    n = b * s
    x = inputs.reshape(n, d)
    idx = labels.reshape(n).astype(jnp.int32)
    out = _make_kernel(n)(x, idx, embedding)
    return out.reshape(b, s, d)
